# Initial kernel scaffold; baseline (speedup 1.0000x reference)
import jax
import jax.numpy as jnp
from jax.experimental import pallas as pl


def kernel(attention_scores):
    raise NotImplementedError("write your pallas kernel here")


# speedup vs baseline: 33.4918x; 33.4918x over previous
import functools

import jax
import jax.numpy as jnp
from jax import lax
from jax.experimental import pallas as pl
from jax.experimental.pallas import tpu as pltpu
from jax.experimental.pallas import tpu_sc as plsc

_NC = 2
_NS = 16
_L = 16
_UNROLL = 8


def _decode_f32(c):
    bits = jnp.where(c >= 0, c, c ^ jnp.int32(0x7FFFFFFF))
    t = lax.bitcast_convert_type(bits, jnp.float32)
    return jnp.full((_L,), t, dtype=jnp.float32)


def _row_kernel(row_v, n, k):
    chunk = _L * _UNROLL

    def bs_body(j, p):
        c = p + (jnp.int32(1) << (jnp.int32(31) - j))
        tv = _decode_f32(c)

        def cnt_body(i, acc):
            base = i * chunk
            for u in range(_UNROLL):
                xv = row_v[pl.ds(base + u * _L, _L)]
                acc = acc + jnp.where(xv >= tv, jnp.int32(1), jnp.int32(0))
            return acc

        acc = lax.fori_loop(0, n // chunk, cnt_body,
                            jnp.zeros((_L,), jnp.int32))
        cnt = jnp.sum(acc)
        return jnp.where(cnt >= k, c, p)

    p = lax.fori_loop(0, 32, bs_body, jnp.int32(-(2 ** 31)))
    tv = _decode_f32(p)

    one = jnp.full((_L,), 1.0, dtype=jnp.float32)
    zero = jnp.zeros((_L,), dtype=jnp.float32)

    def mask_body(i, _):
        base = i * chunk
        for u in range(_UNROLL):
            sl = pl.ds(base + u * _L, _L)
            row_v[sl] = jnp.where(row_v[sl] >= tv, one, zero)
        return 0

    lax.fori_loop(0, n // chunk, mask_body, 0)


def _make_sc_kernel(b, n, k):
    rows_per_w = b // (_NC * _NS)
    mesh = plsc.VectorSubcoreMesh(core_axis_name="c", subcore_axis_name="s")

    @functools.partial(
        pl.kernel,
        out_type=jax.ShapeDtypeStruct((b, n), jnp.float32),
        mesh=mesh,
        scratch_types=[pltpu.VMEM((n,), jnp.float32)],
        compiler_params=pltpu.CompilerParams(needs_layout_passes=False),
    )
    def sc_kernel(x_hbm, out_hbm, row_v):
        wid = lax.axis_index("s") * _NC + lax.axis_index("c")
        for r in range(rows_per_w):
            row = wid * rows_per_w + r
            pltpu.sync_copy(x_hbm.at[row], row_v)
            _row_kernel(row_v, n, k)
            pltpu.sync_copy(row_v, out_hbm.at[row])

    return sc_kernel


@jax.jit
def kernel(attention_scores):
    b, n = attention_scores.shape
    k = max(1, int(n * 0.3))
    return _make_sc_kernel(b, n, k)(attention_scores)
